# baseline (speedup 1.0000x reference)
import functools

import jax
import jax.numpy as jnp
from jax import lax
from jax.experimental import pallas as pl
from jax.experimental.pallas import tpu as pltpu
from jax.experimental.pallas import tpu_sc as plsc

VOCAB = 100000
EMB = 32
TOK = 8
NUMF = 48
OUTW = TOK * EMB + NUMF
LANES = 16
GW = 128
NRING = 4


@functools.lru_cache(maxsize=None)
def _build(n_batch, n_turn):
    info = plsc.get_sparse_core_info()
    nw = info.num_cores * info.num_subcores
    n_g = n_batch // GW
    n_units = n_turn * n_g
    mesh = plsc.VectorSubcoreMesh(core_axis_name="c", subcore_axis_name="s")

    @functools.partial(
        pl.kernel,
        mesh=mesh,
        out_type=jax.ShapeDtypeStruct((n_turn, OUTW, n_batch), jnp.float32),
        scratch_types=[
            pltpu.VMEM((2, TOK, GW), jnp.int32),
            pltpu.VMEM((NRING, GW, 128), jnp.float32),
            pltpu.VMEM((OUTW, GW), jnp.float32),
            pltpu.SemaphoreType.DMA,
            pltpu.SemaphoreType.DMA,
            pltpu.SemaphoreType.DMA,
            pltpu.SemaphoreType.DMA,
        ],
        compiler_params=pltpu.CompilerParams(needs_layout_passes=False),
    )
    def k(idx_hbm, num_hbm, table_hbm, out_hbm,
          idx_v, ring_v, stage_v,
          sem_idx, sem_num, sem_g, sem_w):
        wid = lax.axis_index("s") * info.num_cores + lax.axis_index("c")
        base = n_units // nw
        my_n = base + jnp.where(wid < (n_units - base * nw), 1, 0)

        def unit_tg(u):
            return u // n_g, lax.rem(u, n_g)

        t0, g0 = unit_tg(wid)
        pltpu.async_copy(
            idx_hbm.at[t0, :, pl.ds(g0 * GW, GW)], idx_v.at[0], sem_idx
        )

        def body(i, carry):
            u = wid + i * nw
            t, g = unit_tg(u)
            ib = lax.rem(i, 2)
            pltpu.make_async_copy(
                idx_hbm.at[t, :, pl.ds(g * GW, GW)], idx_v.at[ib], sem_idx
            ).wait()
            for f in range(TOK):
                for o in range(0, GW, LANES):
                    v = idx_v[ib, f, pl.ds(o, LANES)]
                    idx_v[ib, f, pl.ds(o, LANES)] = jnp.clip(
                        v + 1, 0, VOCAB - 1
                    )
            gathers = [None] * TOK
            for f in range(NRING):
                gathers[f] = pltpu.async_copy(
                    table_hbm.at[idx_v.at[ib, f]], ring_v.at[f], sem_g
                )
            @pl.when(i + 1 < my_n)
            def _():
                un = wid + (i + 1) * nw
                tn, gn = unit_tg(un)
                pltpu.async_copy(
                    idx_hbm.at[tn, :, pl.ds(gn * GW, GW)],
                    idx_v.at[1 - ib],
                    sem_idx,
                )

            @pl.when(i > 0)
            def _():
                pltpu.make_async_copy(
                    stage_v, out_hbm.at[t, :, pl.ds(g * GW, GW)], sem_w
                ).wait()

            pltpu.async_copy(
                num_hbm.at[t, :, pl.ds(g * GW, GW)],
                stage_v.at[pl.ds(TOK * EMB, NUMF)],
                sem_num,
            )
            lane = lax.iota(jnp.int32, LANES)
            for f in range(TOK):
                gathers[f].wait()

                @plsc.parallel_loop(0, GW, 1, unroll=8)
                def _(l, _f=f):
                    col = jnp.full((LANES,), 0, jnp.int32) + l
                    for h in range(0, EMB, LANES):
                        v = ring_v[_f % NRING, l, pl.ds(h, LANES)]
                        plsc.store_scatter(
                            stage_v, [lane + (_f * EMB + h), col], v
                        )

                if f + NRING < TOK:
                    gathers[f + NRING] = pltpu.async_copy(
                        table_hbm.at[idx_v.at[ib, f + NRING]],
                        ring_v.at[f % NRING],
                        sem_g,
                    )
            pltpu.make_async_copy(
                num_hbm.at[t, :, pl.ds(g * GW, GW)],
                stage_v.at[pl.ds(TOK * EMB, NUMF)],
                sem_num,
            ).wait()
            pltpu.async_copy(
                stage_v, out_hbm.at[t, :, pl.ds(g * GW, GW)], sem_w
            )
            return carry

        lax.fori_loop(0, my_n, body, 0)
        ul = wid + (my_n - 1) * nw
        tl, gl = unit_tg(ul)
        pltpu.make_async_copy(
            stage_v, out_hbm.at[tl, :, pl.ds(gl * GW, GW)], sem_w
        ).wait()

    return k


def kernel(token_inputs, numerical_inputs, text_emb_table):
    B, Tn, F = token_inputs.shape
    idx_t = token_inputs.astype(jnp.int32).transpose(1, 2, 0)
    num_t = numerical_inputs.transpose(1, 2, 0)
    table_p = jnp.pad(text_emb_table, ((0, 0), (0, 128 - EMB)))
    out_t = _build(B, Tn)(idx_t, num_t, table_p)
    return out_t.transpose(2, 0, 1)


# speedup vs baseline: 1.0074x; 1.0074x over previous
import functools

import jax
import jax.numpy as jnp
from jax import lax
from jax.experimental import pallas as pl
from jax.experimental.pallas import tpu as pltpu
from jax.experimental.pallas import tpu_sc as plsc

VOCAB = 100000
EMB = 32
TOK = 8
NUMF = 48
OUTW = TOK * EMB + NUMF
LANES = 16
GW = 100
NG = 4
T = 50
HALF = NG * GW // 2


@functools.lru_cache(maxsize=None)
def _build(n_batch):
    info = plsc.get_sparse_core_info()
    nw = info.num_cores * info.num_subcores
    per_w = n_batch // nw
    assert n_batch % nw == 0

    mesh = plsc.VectorSubcoreMesh(core_axis_name="c", subcore_axis_name="s")

    CLIP_OFFS = (0, 16, 32, 48, 64, 80)

    @functools.partial(
        pl.kernel,
        mesh=mesh,
        out_type=jax.ShapeDtypeStruct((n_batch, T, OUTW), jnp.float32),
        scratch_types=[
            pltpu.VMEM((2, NG, GW), jnp.int32),
            pltpu.VMEM((2, HALF, 128), jnp.float32),
            pltpu.VMEM((T, NUMF), jnp.float32),
            pltpu.VMEM((T, OUTW), jnp.float32),
            pltpu.SemaphoreType.DMA,
            pltpu.SemaphoreType.DMA,
            pltpu.SemaphoreType.DMA,
            pltpu.SemaphoreType.DMA,
        ],
    )
    def k(idx_hbm, num_hbm, table_hbm, out_hbm,
          idx_v, pad_v, num_v, stage_v,
          sem_idx, sem_num, sem_g, sem_w):
        wid = lax.axis_index("s") * info.num_cores + lax.axis_index("c")
        b0 = wid * per_w

        pltpu.async_copy(idx_hbm.at[b0], idx_v.at[0], sem_idx)
        pltpu.async_copy(num_hbm.at[b0], num_v, sem_num)

        def body(i, carry):
            b = b0 + i
            ib = lax.rem(i, 2)
            pltpu.make_async_copy(idx_hbm.at[b], idx_v.at[ib], sem_idx).wait()
            lane = lax.iota(jnp.int32, LANES)
            for r in range(NG):
                for o in CLIP_OFFS:
                    v = idx_v[ib, r, pl.ds(o, LANES)]
                    idx_v[ib, r, pl.ds(o, LANES)] = jnp.clip(
                        v + 1, 0, VOCAB - 1
                    )
                v = idx_v[ib, r, pl.ds(GW - LANES, LANES)]
                idx_v[ib, r, pl.ds(GW - LANES, LANES)] = jnp.where(
                    lane < (96 - (GW - LANES)),
                    v,
                    jnp.clip(v + 1, 0, VOCAB - 1),
                )
            gathers = [
                pltpu.async_copy(
                    table_hbm.at[idx_v.at[ib, r]],
                    pad_v.at[r // 2, pl.ds((r % 2) * GW, GW)],
                    sem_g,
                )
                for r in range(NG)
            ]
            @pl.when(i + 1 < per_w)
            def _():
                pltpu.async_copy(
                    idx_hbm.at[b + 1], idx_v.at[1 - ib], sem_idx
                )

            @pl.when(i > 0)
            def _():
                pltpu.make_async_copy(stage_v, out_hbm.at[b], sem_w).wait()

            pltpu.make_async_copy(num_hbm.at[b], num_v, sem_num).wait()
            for j in range(T):
                for h in range(NUMF // LANES):
                    stage_v[j, pl.ds(TOK * EMB + h * LANES, LANES)] = num_v[
                        j, pl.ds(h * LANES, LANES)
                    ]
            for half in range(2):
                gathers[2 * half].wait()
                gathers[2 * half + 1].wait()
                for t in range(HALF):
                    n = half * HALF + t
                    turn, tok = n // TOK, n % TOK
                    for h in range(EMB // LANES):
                        stage_v[
                            turn, pl.ds(tok * EMB + h * LANES, LANES)
                        ] = pad_v[half, t, pl.ds(h * LANES, LANES)]
            pltpu.async_copy(stage_v, out_hbm.at[b], sem_w)

            @pl.when(i + 1 < per_w)
            def _():
                pltpu.async_copy(num_hbm.at[b + 1], num_v, sem_num)

            return carry

        lax.fori_loop(0, per_w, body, 0)
        pltpu.make_async_copy(
            stage_v, out_hbm.at[b0 + per_w - 1], sem_w
        ).wait()

    return k


def kernel(token_inputs, numerical_inputs, text_emb_table):
    B, Tn, F = token_inputs.shape
    idx = token_inputs.astype(jnp.int32).reshape(B, NG, GW)
    table_p = jax.lax.dynamic_update_slice(
        jnp.zeros((VOCAB, 128), jnp.float32), text_emb_table, (0, 0)
    )
    return _build(B)(idx, numerical_inputs, table_p)
